# baseline (speedup 1.0000x reference)
import functools

import jax
import jax.numpy as jnp
from jax import lax
from jax.experimental import pallas as pl
from jax.experimental.pallas import tpu as pltpu
from jax.experimental.pallas import tpu_sc as plsc


def _build(B, F, V):
    info = plsc.get_sparse_core_info()
    NC, NS, L = info.num_cores, info.num_subcores, info.num_lanes
    NW = NC * NS
    bpw = B // NW
    groups = bpw // L
    FP = 32

    mesh = plsc.VectorSubcoreMesh(core_axis_name="c", subcore_axis_name="s")

    @functools.partial(
        pl.kernel,
        out_type=jax.ShapeDtypeStruct((B,), jnp.float32),
        mesh=mesh,
        compiler_params=pltpu.CompilerParams(
            needs_layout_passes=False, use_tc_tiling_on_sc=True),
        scratch_types=[
            pltpu.VMEM((V,), jnp.float32),
            pltpu.VMEM((F, bpw), jnp.int32),
            pltpu.VMEM((bpw,), jnp.float32),
            pltpu.VMEM((FP,), jnp.int32),
            pltpu.SemaphoreType.DMA,
            pltpu.SemaphoreType.DMA,
        ],
    )
    def k(xt_hbm, tab_hbm, off_hbm, out_hbm,
          tab_v, x_v, o_v, off_v, sem_t, sem_x):
        wid = lax.axis_index("s") * NC + lax.axis_index("c")
        cp_t = pltpu.async_copy(tab_hbm, tab_v, sem_t)
        cp_x = pltpu.async_copy(xt_hbm.at[:, pl.ds(wid * bpw, bpw)], x_v, sem_x)
        pltpu.sync_copy(off_hbm, off_v)

        bias_vec = plsc.bitcast(
            plsc.load_gather(off_v, [jnp.full((L,), FP - 1, jnp.int32)]),
            jnp.float32)
        off_vecs = [
            plsc.load_gather(off_v, [jnp.full((L,), f + 1, jnp.int32)])
            for f in range(F)
        ]

        cp_x.wait()
        cp_t.wait()

        def body(g, carry):
            col = g * L
            acc = bias_vec
            for f in range(F):
                idx = x_v[f, pl.ds(col, L)] + off_vecs[f]
                acc = acc + plsc.load_gather(tab_v, [idx])
            o_v[pl.ds(col, L)] = acc
            return carry

        lax.fori_loop(0, groups, body, 0, unroll=8)
        pltpu.sync_copy(o_v, out_hbm.at[pl.ds(wid * bpw, bpw)])

    return k


def kernel(x, table, offsets, bias):
    B, F = x.shape
    V = table.shape[0]
    bias_bits = jax.lax.bitcast_convert_type(bias.astype(jnp.float32), jnp.int32)
    pack = (jnp.zeros((32,), jnp.int32)
            .at[1:F + 1].set(offsets.astype(jnp.int32))
            .at[31].set(bias_bits[0]))
    out = _build(B, F, V)(x.astype(jnp.int32).T, table.reshape(-1), pack)
    return out[:, None]


# speedup vs baseline: 1.0108x; 1.0108x over previous
import functools

import jax
import jax.numpy as jnp
from jax import lax
from jax.experimental import pallas as pl
from jax.experimental.pallas import tpu as pltpu
from jax.experimental.pallas import tpu_sc as plsc


def _build(B, F, V):
    info = plsc.get_sparse_core_info()
    NC, NS, L = info.num_cores, info.num_subcores, info.num_lanes
    NW = NC * NS
    bpw = B // NW
    groups = bpw // L
    FP = 32

    mesh = plsc.VectorSubcoreMesh(core_axis_name="c", subcore_axis_name="s")

    @functools.partial(
        pl.kernel,
        out_type=jax.ShapeDtypeStruct((B,), jnp.float32),
        mesh=mesh,
        compiler_params=pltpu.CompilerParams(
            needs_layout_passes=False, use_tc_tiling_on_sc=True),
        scratch_types=[
            pltpu.VMEM((V,), jnp.float32),
            pltpu.VMEM((F, bpw), jnp.int32),
            pltpu.VMEM((bpw,), jnp.float32),
            pltpu.VMEM((FP,), jnp.int32),
            pltpu.SemaphoreType.DMA,
            pltpu.SemaphoreType.DMA,
        ],
    )
    def k(xt_hbm, tab_hbm, off_hbm, out_hbm,
          tab_v, x_v, o_v, off_v, sem_t, sem_x):
        wid = lax.axis_index("s") * NC + lax.axis_index("c")
        cp_t = pltpu.async_copy(tab_hbm, tab_v, sem_t)
        cp_x = pltpu.async_copy(xt_hbm.at[:, pl.ds(wid * bpw, bpw)], x_v, sem_x)
        pltpu.sync_copy(off_hbm, off_v)

        bias_vec = plsc.bitcast(
            plsc.load_gather(off_v, [jnp.full((L,), FP - 1, jnp.int32)]),
            jnp.float32)
        off_vecs = [
            plsc.load_gather(off_v, [jnp.full((L,), f + 1, jnp.int32)])
            for f in range(F)
        ]

        cp_x.wait()
        cp_t.wait()

        def body(g, carry):
            col = g * L
            acc = bias_vec
            for f in range(F):
                idx = x_v[f, pl.ds(col, L)] + off_vecs[f]
                acc = acc + plsc.load_gather(tab_v, [idx])
            o_v[pl.ds(col, L)] = acc
            return carry

        lax.fori_loop(0, groups, body, 0, unroll=4)
        pltpu.sync_copy(o_v, out_hbm.at[pl.ds(wid * bpw, bpw)])

    return k


def kernel(x, table, offsets, bias):
    B, F = x.shape
    V = table.shape[0]
    bias_bits = jax.lax.bitcast_convert_type(bias.astype(jnp.float32), jnp.int32)
    pack = (jnp.zeros((32,), jnp.int32)
            .at[1:F + 1].set(offsets.astype(jnp.int32))
            .at[31].set(bias_bits[0]))
    out = _build(B, F, V)(x.astype(jnp.int32).T, table.reshape(-1), pack)
    return out[:, None]
